# baseline (speedup 1.0000x reference)
import functools

import jax
import jax.numpy as jnp
from jax import lax
from jax.experimental import pallas as pl
from jax.experimental.pallas import tpu as pltpu
from jax.experimental.pallas import tpu_sc as plsc

BANK = 1_000_000
EMB = 32
BS = 16384

NCORES = 2
NSUB = 16
NW = NCORES * NSUB
CH = 480
NCH = 65
SLAB = CH * NCH
TAILB = NW * SLAB
NTAIL = 4
NBUF = 2
DPRE = 1

_mesh = plsc.VectorSubcoreMesh(core_axis_name="c", subcore_axis_name="s")


@functools.partial(
    pl.kernel,
    out_type=jax.ShapeDtypeStruct((BANK, EMB), jnp.float32),
    mesh=_mesh,
    scratch_types=[
        pltpu.VMEM_SHARED((NSUB, NBUF, CH, EMB), jnp.float32),
        pltpu.SemaphoreType.DMA((NBUF,)),
        pltpu.SemaphoreType.DMA((NBUF,)),
    ],
)
def _sc_copy(q_hbm, out_hbm, bufs, lsem, ssem):
    cid = lax.axis_index("c")
    sid = lax.axis_index("s")
    wid = sid * NCORES + cid
    base = pl.multiple_of(wid * SLAB, 8)

    def load(c, b):
        return pltpu.make_async_copy(
            q_hbm.at[pl.ds(pl.multiple_of(base + c * CH, 8), CH), :],
            bufs.at[sid, b], lsem.at[b])

    def store(c, b):
        return pltpu.make_async_copy(
            bufs.at[sid, b],
            out_hbm.at[pl.ds(pl.multiple_of(base + c * CH, 8), CH), :],
            ssem.at[b])

    for c in range(-DPRE, NCH):
        if c >= 0:
            b = c % NBUF
            load(c, b).wait()
            store(c, b).start()
        n = c + DPRE
        if 0 <= n < NCH:
            m = n - NBUF
            if m >= 0:
                store(m, m % NBUF).wait()
            load(n, n % NBUF).start()
    for c in range(max(NCH - NBUF, 0), NCH):
        store(c, c % NBUF).wait()

    @pl.when(wid == 0)
    def _():
        for t in range(NTAIL):
            n = min(CH, BANK - (TAILB + t * CH))
            cp = pltpu.make_async_copy(
                q_hbm.at[pl.ds(TAILB + t * CH, n), :],
                bufs.at[sid, 0, pl.ds(0, n), :], lsem.at[0])
            cp.start()
            cp.wait()
            cp2 = pltpu.make_async_copy(
                bufs.at[sid, 0, pl.ds(0, n), :],
                out_hbm.at[pl.ds(TAILB + t * CH, n), :], ssem.at[0])
            cp2.start()
            cp2.wait()


WB = 4_000
NB = BANK // WB
NWIN = BS // WB + 2
EPAD = BS + 2 * WB


def _win_body(ptr_ref, emb_ref, q_ref, out_ref):
    i = pl.program_id(0)
    p = ptr_ref[0]
    s = (jax.lax.rem(p // WB + i, NB)) * WB

    o = jax.lax.rem(s - p + BANK, BANK)
    b = jnp.where(o >= BANK - WB, o - BANK, o)
    b = jnp.clip(b, -WB, BS)
    emb_slice = emb_ref[pl.ds(b + WB, WB), :]

    j = jax.lax.broadcasted_iota(jnp.int32, (WB, 1), 0)
    d0 = o + j
    delta = jnp.where(d0 >= BANK, d0 - BANK, d0)
    take = delta < BS
    out_ref[:, :] = jnp.where(take, emb_slice, q_ref[:, :])


def _win_update(p, emb_p, q):
    grid_spec = pltpu.PrefetchScalarGridSpec(
        num_scalar_prefetch=1,
        grid=(NWIN,),
        in_specs=[
            pl.BlockSpec((EPAD, EMB), lambda i, pr: (0, 0)),
            pl.BlockSpec((WB, EMB),
                         lambda i, pr: (jax.lax.rem(pr[0] // WB + i, NB), 0)),
        ],
        out_specs=pl.BlockSpec((WB, EMB),
                               lambda i, pr: (jax.lax.rem(pr[0] // WB + i, NB), 0)),
    )
    return pl.pallas_call(
        _win_body,
        grid_spec=grid_spec,
        out_shape=jax.ShapeDtypeStruct((BANK, EMB), jnp.float32),
        input_output_aliases={2: 0},
    )(p, emb_p, q)


def kernel(embeddings, queue, ptr):
    p = jax.lax.rem(jnp.asarray(ptr, jnp.int32), BANK).reshape(1)
    emb_p = jnp.pad(embeddings, ((WB, WB), (0, 0)))
    qc = _sc_copy(queue)
    return _win_update(p, emb_p, qc)


# speedup vs baseline: 1.5616x; 1.5616x over previous
import jax
import jax.numpy as jnp
from jax.experimental import pallas as pl
from jax.experimental.pallas import tpu as pltpu

BANK = 1_000_000
EMB = 32
BS = 16384
WB = 4_000
NB = BANK // WB
NWIN = BS // WB + 2
EPAD = BS + 2 * WB


def _win_body(ptr_ref, emb_ref, q_ref, out_ref):
    i = pl.program_id(0)
    p = ptr_ref[0]
    s = (jax.lax.rem(p // WB + i, NB)) * WB

    o = jax.lax.rem(s - p + BANK, BANK)
    b = jnp.where(o >= BANK - WB, o - BANK, o)
    b = jnp.clip(b, -WB, BS)
    emb_slice = emb_ref[pl.ds(b + WB, WB), :]

    j = jax.lax.broadcasted_iota(jnp.int32, (WB, 1), 0)
    d0 = o + j
    delta = jnp.where(d0 >= BANK, d0 - BANK, d0)
    take = delta < BS
    out_ref[:, :] = jnp.where(take, emb_slice, q_ref[:, :])


def kernel(embeddings, queue, ptr):
    p = jax.lax.rem(jnp.asarray(ptr, jnp.int32), BANK).reshape(1)
    emb_p = jnp.pad(embeddings, ((WB, WB), (0, 0)))
    grid_spec = pltpu.PrefetchScalarGridSpec(
        num_scalar_prefetch=1,
        grid=(NWIN,),
        in_specs=[
            pl.BlockSpec((EPAD, EMB), lambda i, pr: (0, 0)),
            pl.BlockSpec((WB, EMB),
                         lambda i, pr: (jax.lax.rem(pr[0] // WB + i, NB), 0)),
        ],
        out_specs=pl.BlockSpec((WB, EMB),
                               lambda i, pr: (jax.lax.rem(pr[0] // WB + i, NB), 0)),
    )
    return pl.pallas_call(
        _win_body,
        grid_spec=grid_spec,
        out_shape=jax.ShapeDtypeStruct((BANK, EMB), jnp.float32),
        input_output_aliases={2: 0},
    )(p, emb_p, queue)
